# baseline (device time: 28913 ns/iter reference)
import jax
import jax.numpy as jnp
from jax import lax
from jax.experimental import pallas as pl
from jax.experimental.pallas import tpu as pltpu

T = 512
D = 1024
V_LOCAL = 8192
V_CHUNK = 2048
N_CHUNKS = V_LOCAL // V_CHUNK
NZ = 4

_CompilerParams = getattr(pltpu, "CompilerParams", None) or getattr(
    pltpu, "TPUCompilerParams"
)


def kernel(x, W, labels):
    def body(xt_ref, w_ref, lab_ref, out_ref, rowstats, comm_ref,
             send_sems, recv_sems):
        j = pl.program_id(0)
        my_x = lax.axis_index("x")
        my_y = lax.axis_index("y")
        my_z = lax.axis_index("z")
        barrier = pltpu.get_barrier_semaphore()

        @pl.when(j == 0)
        def _():
            for dz in range(1, NZ):
                pz = lax.rem(my_z + dz, NZ)
                pl.semaphore_signal(
                    barrier, inc=1,
                    device_id=(my_x, my_y, pz),
                    device_id_type=pl.DeviceIdType.MESH,
                )
            pl.semaphore_wait(barrier, NZ - 1)

        logits_t = lax.dot_general(
            w_ref[:, :].astype(jnp.bfloat16),
            xt_ref[:, :].astype(jnp.bfloat16),
            dimension_numbers=(((0,), (0,)), ((), ())),
            preferred_element_type=jnp.float32,
        )

        s_part = jnp.sum(jnp.exp(logits_t), axis=0, keepdims=True)
        labloc = lab_ref[:, :] - (my_z * V_LOCAL + j * V_CHUNK)
        vids = lax.broadcasted_iota(jnp.int32, (V_CHUNK, T), 0)
        contrib = jnp.sum(
            jnp.where(vids == labloc, logits_t, 0.0),
            axis=0, keepdims=True,
        )

        @pl.when(j == 0)
        def _():
            rowstats[0:1, :] = s_part
            rowstats[1:2, :] = contrib

        @pl.when(j != 0)
        def _():
            rowstats[0:1, :] = rowstats[0:1, :] + s_part
            rowstats[1:2, :] = rowstats[1:2, :] + contrib

        @pl.when(j == N_CHUNKS - 1)
        def _():
            sends = []
            for dz in range(1, NZ):
                pz = lax.rem(my_z + dz, NZ)
                rdma = pltpu.make_async_remote_copy(
                    src_ref=rowstats,
                    dst_ref=comm_ref.at[NZ - dz],
                    send_sem=send_sems.at[dz - 1],
                    recv_sem=recv_sems.at[NZ - dz],
                    device_id=(my_x, my_y, pz),
                    device_id_type=pl.DeviceIdType.MESH,
                )
                rdma.start()
                sends.append(rdma)

            s_g = rowstats[0:1, :]
            lbl = rowstats[1:2, :]
            for slot in range(1, NZ):
                recv = pltpu.make_async_remote_copy(
                    src_ref=rowstats,
                    dst_ref=comm_ref.at[slot],
                    send_sem=send_sems.at[0],
                    recv_sem=recv_sems.at[slot],
                    device_id=(my_x, my_y, my_z),
                    device_id_type=pl.DeviceIdType.MESH,
                )
                recv.wait_recv()
                blk = comm_ref[slot]
                s_g = s_g + blk[0:1, :]
                lbl = lbl + blk[1:2, :]

            out_ref[:, :] = jnp.log(s_g) - lbl

            for s in sends:
                s.wait_send()

    out = pl.pallas_call(
        body,
        grid=(N_CHUNKS,),
        in_specs=[
            pl.BlockSpec((D, T), lambda j: (0, 0)),
            pl.BlockSpec((D, V_CHUNK), lambda j: (0, j)),
            pl.BlockSpec((1, T), lambda j: (0, 0)),
        ],
        out_specs=pl.BlockSpec((1, T), lambda j: (0, 0)),
        out_shape=jax.ShapeDtypeStruct((1, T), jnp.float32),
        scratch_shapes=[
            pltpu.VMEM((8, T), jnp.float32),
            pltpu.VMEM((NZ, 8, T), jnp.float32),
            pltpu.SemaphoreType.DMA((NZ - 1,)),
            pltpu.SemaphoreType.DMA((NZ,)),
        ],
        compiler_params=_CompilerParams(
            dimension_semantics=("arbitrary",),
            collective_id=0,
            vmem_limit_bytes=60 * 1024 * 1024,
        ),
    )(x.T, W, labels.reshape(1, T))
    return out.reshape(T)


# device time: 25915 ns/iter; 1.1157x vs baseline; 1.1157x over previous
import jax
import jax.numpy as jnp
from jax import lax
from jax.experimental import pallas as pl
from jax.experimental.pallas import tpu as pltpu

T = 512
D = 1024
V_LOCAL = 8192
V_CHUNK = 2048
N_CHUNKS = V_LOCAL // V_CHUNK
NZ = 4

_CompilerParams = getattr(pltpu, "CompilerParams", None) or getattr(
    pltpu, "TPUCompilerParams"
)


def kernel(x, W, labels):
    def body(x_ref, w_ref, lab_ref, out_ref, colstats, rowstats,
             comm_ref, send_sems, recv_sems):
        j = pl.program_id(0)
        my_x = lax.axis_index("x")
        my_y = lax.axis_index("y")
        my_z = lax.axis_index("z")
        barrier = pltpu.get_barrier_semaphore()

        @pl.when(j == 0)
        def _():
            for dz in range(1, NZ):
                pz = lax.rem(my_z + dz, NZ)
                pl.semaphore_signal(
                    barrier, inc=1,
                    device_id=(my_x, my_y, pz),
                    device_id_type=pl.DeviceIdType.MESH,
                )

        logits = lax.dot_general(
            x_ref[:, :].astype(jnp.bfloat16),
            w_ref[:, :].astype(jnp.bfloat16),
            dimension_numbers=(((1,), (0,)), ((), ())),
            preferred_element_type=jnp.float32,
        )

        s_part = jnp.sum(jnp.exp(logits), axis=1, keepdims=True)
        labloc = lab_ref[:, :] - (my_z * V_LOCAL + j * V_CHUNK)
        vids = lax.broadcasted_iota(jnp.int32, (T, V_CHUNK), 1)
        contrib = jnp.sum(
            jnp.where(vids == labloc, logits, 0.0),
            axis=1, keepdims=True,
        )

        @pl.when(j == 0)
        def _():
            colstats[:, 0:1] = s_part
            colstats[:, 1:2] = contrib

        @pl.when(j != 0)
        def _():
            colstats[:, 0:1] = colstats[:, 0:1] + s_part
            colstats[:, 1:2] = colstats[:, 1:2] + contrib

        @pl.when(j == N_CHUNKS - 1)
        def _():
            r = lax.broadcasted_iota(jnp.int32, (T, T), 0)
            c = lax.broadcasted_iota(jnp.int32, (T, T), 1)
            eye = jnp.where(r == c, 1.0, 0.0).astype(jnp.float32)
            rowstats[:, :] = lax.dot_general(
                colstats[:, :], eye,
                dimension_numbers=(((0,), (0,)), ((), ())),
                preferred_element_type=jnp.float32,
            )

            pl.semaphore_wait(barrier, NZ - 1)

            sends = []
            for dz in range(1, NZ):
                pz = lax.rem(my_z + dz, NZ)
                rdma = pltpu.make_async_remote_copy(
                    src_ref=rowstats,
                    dst_ref=comm_ref.at[NZ - dz],
                    send_sem=send_sems.at[dz - 1],
                    recv_sem=recv_sems.at[NZ - dz],
                    device_id=(my_x, my_y, pz),
                    device_id_type=pl.DeviceIdType.MESH,
                )
                rdma.start()
                sends.append(rdma)

            s_g = rowstats[0:1, :]
            lbl = rowstats[1:2, :]
            for slot in range(1, NZ):
                recv = pltpu.make_async_remote_copy(
                    src_ref=rowstats,
                    dst_ref=comm_ref.at[slot],
                    send_sem=send_sems.at[0],
                    recv_sem=recv_sems.at[slot],
                    device_id=(my_x, my_y, my_z),
                    device_id_type=pl.DeviceIdType.MESH,
                )
                recv.wait_recv()
                blk = comm_ref[slot]
                s_g = s_g + blk[0:1, :]
                lbl = lbl + blk[1:2, :]

            out_ref[:, :] = jnp.log(s_g) - lbl

            for s in sends:
                s.wait_send()

    out = pl.pallas_call(
        body,
        grid=(N_CHUNKS,),
        in_specs=[
            pl.BlockSpec((T, D), lambda j: (0, 0)),
            pl.BlockSpec((D, V_CHUNK), lambda j: (0, j)),
            pl.BlockSpec((T, 1), lambda j: (0, 0)),
        ],
        out_specs=pl.BlockSpec((1, T), lambda j: (0, 0)),
        out_shape=jax.ShapeDtypeStruct((1, T), jnp.float32),
        scratch_shapes=[
            pltpu.VMEM((T, 8), jnp.float32),
            pltpu.VMEM((8, T), jnp.float32),
            pltpu.VMEM((NZ, 8, T), jnp.float32),
            pltpu.SemaphoreType.DMA((NZ - 1,)),
            pltpu.SemaphoreType.DMA((NZ,)),
        ],
        compiler_params=_CompilerParams(
            dimension_semantics=("arbitrary",),
            collective_id=0,
            vmem_limit_bytes=60 * 1024 * 1024,
        ),
    )(x, W, labels.reshape(T, 1))
    return out.reshape(T)
